# baseline (speedup 1.0000x reference)
import jax
import jax.numpy as jnp
from jax import lax
from jax.experimental import pallas as pl
from jax.experimental.pallas import tpu as pltpu

_EPS = 1e-5


def _make_body(cin, cout, ho, wo, tr, n_row_tiles):
    inv_s = 1.0 / float(ho * wo)
    tw = tr * wo
    c2 = 2 * cin
    wp = 2 * wo

    def body(xm_ref, xb_ref, sel_ref, w_ref, o_ref,
             pln_ref, conv_ref, sums_ref, stash_ref):
        t = pl.program_id(1)

        @pl.when(t < n_row_tiles)
        def _conv_phase():
            @pl.when(t == 0)
            def _():
                sums_ref[...] = jnp.zeros_like(sums_ref)

            sel = sel_ref[...]
            xmb = xm_ref[0].astype(jnp.bfloat16)
            xf = jnp.swapaxes(xmb, 0, 1).reshape(2 * tr * cin, wp)
            e3 = jnp.dot(xf, sel,
                         preferred_element_type=jnp.float32).astype(jnp.bfloat16)
            top = jnp.where(t == 0, xmb[:, 1, :], stash_ref[...])
            stash_ref[...] = xmb[:, 2 * tr - 1, :]
            bot = jnp.where(t == n_row_tiles - 1,
                            xb_ref[0, :, 6, :], xb_ref[0, :, 0, :]).astype(jnp.bfloat16)
            top3 = jnp.dot(top, sel,
                           preferred_element_type=jnp.float32).astype(jnp.bfloat16)
            bot3 = jnp.dot(bot, sel,
                           preferred_element_type=jnp.float32).astype(jnp.bfloat16)
            for i in range(tr + 1):
                s = slice(i * wo, (i + 1) * wo)
                s0 = top3 if i == 0 else e3[(2 * i - 1) * cin:2 * i * cin]
                pln_ref[0 * cin:1 * cin, s] = s0[:, 0:wo]
                pln_ref[1 * cin:2 * cin, s] = s0[:, wo + 1:2 * wo + 1]
                pln_ref[4 * cin:5 * cin, s] = s0[:, 1:wo + 1]
                s1 = bot3 if i == tr else e3[2 * i * cin:(2 * i + 1) * cin]
                pln_ref[2 * cin:3 * cin, s] = s1[:, 0:wo]
                pln_ref[3 * cin:4 * cin, s] = s1[:, wo + 1:2 * wo + 1]
                pln_ref[5 * cin:6 * cin, s] = s1[:, 1:wo + 1]

            a0 = pln_ref[0:2 * c2, :]
            a1 = pln_ref[2 * c2:3 * c2, :]
            xs = (
                a0[0:c2, 0:tw],
                a0[c2:2 * c2, 0:tw],
                a0[0:c2, wo:tw + wo],
                a1[:, 0:tw],
                a1[:, wo:tw + wo],
            )
            acc = jnp.zeros((cout, tw), jnp.float32)
            for i, x in enumerate(xs):
                acc = acc + jnp.dot(w_ref[i], x,
                                    preferred_element_type=jnp.float32)
            conv_ref[:, pl.ds(t * tw, tw)] = acc.astype(jnp.bfloat16)
            sums_ref[0] = sums_ref[0] + jnp.sum(acc, axis=1, keepdims=True)
            sums_ref[1] = sums_ref[1] + jnp.sum(acc * acc, axis=1, keepdims=True)

        @pl.when(t >= n_row_tiles)
        def _norm_phase():
            mean = sums_ref[0] * inv_s
            var = sums_ref[1] * inv_s - mean * mean
            rstd = lax.rsqrt(var + _EPS)
            c0 = pl.multiple_of((t - n_row_tiles) * tw, tw)
            tile = conv_ref[:, pl.ds(c0, tw)]
            y = jnp.maximum((tile - mean) * rstd, 0.0).astype(o_ref.dtype)
            o_ref[0] = y.reshape(cout, tr, wo)

    return body


def kernel(x, weight, bias=None):
    del bias
    n, cin, h, w = x.shape
    cout = weight.shape[0]
    ho, wo = h // 2, w // 2
    if ho % 32 == 0 and ho > 32:
        tr = 32
    elif ho % 16 == 0 and ho > 16:
        tr = 16
    else:
        tr = 8 if ho % 8 == 0 else ho
    n_row_tiles = ho // tr


    cols = jnp.arange(2 * wo + 1)
    rows = jnp.arange(w)[:, None]
    tgt = jnp.where(cols <= wo,
                    jnp.where(cols == 0, 1, 2 * cols - 1),
                    2 * (cols - wo - 1))
    sel = (rows == tgt[None, :]).astype(jnp.bfloat16)

    wt = [[weight[:, :, kh, kw].astype(jnp.bfloat16) for kw in range(3)]
          for kh in range(3)]
    z = jnp.zeros((cout, cin), jnp.bfloat16)
    wall = jnp.stack([
        jnp.concatenate([wt[0][0], wt[0][1]], axis=1),
        jnp.concatenate([wt[1][0], wt[1][1]], axis=1),
        jnp.concatenate([wt[2][0], wt[2][1]], axis=1),
        jnp.concatenate([wt[0][2], wt[1][2]], axis=1),
        jnp.concatenate([wt[2][2], z], axis=1),
    ], axis=0)

    tw = tr * wo
    nr = n_row_tiles
    body = _make_body(cin, cout, ho, wo, tr, nr)
    out = pl.pallas_call(
        body,
        out_shape=jax.ShapeDtypeStruct((n, cout, ho, wo), x.dtype),
        grid=(n, 2 * nr),
        in_specs=[
            pl.BlockSpec((1, cin, 2 * tr, w),
                         lambda b, t: (b, 0, jnp.minimum(t, nr - 1), 0)),
            pl.BlockSpec((1, cin, 8, w),
                         lambda b, t: (b, 0, jnp.minimum(
                             (tr // 4) * (jnp.minimum(t, nr - 1) + 1),
                             h // 8 - 1), 0)),
            pl.BlockSpec((w, 2 * wo + 1), lambda b, t: (0, 0)),
            pl.BlockSpec((5, cout, 2 * cin), lambda b, t: (0, 0, 0)),
        ],
        out_specs=pl.BlockSpec((1, cout, tr, wo),
                               lambda b, t: (b, 0, jnp.maximum(t - nr, 0), 0)),
        scratch_shapes=[
            pltpu.VMEM((6 * cin, (tr + 1) * wo), jnp.bfloat16),
            pltpu.VMEM((cout, ho * wo), jnp.bfloat16),
            pltpu.VMEM((2, cout, 1), jnp.float32),
            pltpu.VMEM((cin, w), jnp.bfloat16),
        ],
        compiler_params=pltpu.CompilerParams(
            dimension_semantics=("parallel", "arbitrary")),
    )(x, x, sel, wall)
    return out


# speedup vs baseline: 1.0923x; 1.0923x over previous
import jax
import jax.numpy as jnp
from jax import lax
from jax.experimental import pallas as pl
from jax.experimental.pallas import tpu as pltpu

_EPS = 1e-5


def _make_body(cin, cout, ho, wo, tr, n_row_tiles):
    inv_s = 1.0 / float(ho * wo)
    tw = tr * wo
    c2 = 2 * cin
    wp = 2 * wo

    def body(xm_ref, xb_ref, sel_ref, w_ref, o_ref,
             pln_ref, conv_ref, sums_ref, stash_ref):
        t = pl.program_id(1)

        @pl.when(t < n_row_tiles)
        def _conv_phase():
            @pl.when(t == 0)
            def _():
                sums_ref[...] = jnp.zeros_like(sums_ref)

            sel = sel_ref[...]
            xf = jnp.swapaxes(xm_ref[0], 0, 1).reshape(2 * tr * cin, wp)
            e3 = jnp.dot(xf, sel,
                         preferred_element_type=jnp.float32).astype(jnp.bfloat16)
            top = jnp.where(t == 0, xm_ref[0, :, 1, :], stash_ref[...])
            stash_ref[...] = xm_ref[0, :, 2 * tr - 1, :]
            bot = jnp.where(t == n_row_tiles - 1,
                            xb_ref[0, :, 6, :], xb_ref[0, :, 0, :])
            top3 = jnp.dot(top, sel,
                           preferred_element_type=jnp.float32).astype(jnp.bfloat16)
            bot3 = jnp.dot(bot, sel,
                           preferred_element_type=jnp.float32).astype(jnp.bfloat16)
            for i in range(tr + 1):
                s = slice(i * wo, (i + 1) * wo)
                s0 = top3 if i == 0 else e3[(2 * i - 1) * cin:2 * i * cin]
                pln_ref[0 * cin:1 * cin, s] = s0[:, 0:wo]
                pln_ref[1 * cin:2 * cin, s] = s0[:, wo + 1:2 * wo + 1]
                pln_ref[4 * cin:5 * cin, s] = s0[:, 1:wo + 1]
                s1 = bot3 if i == tr else e3[2 * i * cin:(2 * i + 1) * cin]
                pln_ref[2 * cin:3 * cin, s] = s1[:, 0:wo]
                pln_ref[3 * cin:4 * cin, s] = s1[:, wo + 1:2 * wo + 1]
                pln_ref[5 * cin:6 * cin, s] = s1[:, 1:wo + 1]

            a0 = pln_ref[0:2 * c2, :]
            a1 = pln_ref[2 * c2:3 * c2, :]
            xs = (
                a0[0:c2, 0:tw],
                a0[c2:2 * c2, 0:tw],
                a0[0:c2, wo:tw + wo],
                a1[:, 0:tw],
                a1[:, wo:tw + wo],
            )
            acc = jnp.zeros((cout, tw), jnp.float32)
            for i, x in enumerate(xs):
                acc = acc + jnp.dot(w_ref[i], x,
                                    preferred_element_type=jnp.float32)
            conv_ref[:, pl.ds(t * tw, tw)] = acc.astype(jnp.bfloat16)
            sums_ref[0] = sums_ref[0] + jnp.sum(acc, axis=1, keepdims=True)
            sums_ref[1] = sums_ref[1] + jnp.sum(acc * acc, axis=1, keepdims=True)

        @pl.when(t == n_row_tiles)
        def _norm_phase():
            mean = sums_ref[0] * inv_s
            var = sums_ref[1] * inv_s - mean * mean
            rstd = lax.rsqrt(var + _EPS)
            tile = conv_ref[...]
            y = jnp.maximum((tile - mean) * rstd, 0.0).astype(o_ref.dtype)
            o_ref[0] = y.reshape(cout, ho, wo)

    return body


def kernel(x, weight, bias=None):
    del bias
    n, cin, h, w = x.shape
    cout = weight.shape[0]
    ho, wo = h // 2, w // 2
    if ho % 32 == 0 and ho > 32:
        tr = 32
    elif ho % 16 == 0 and ho > 16:
        tr = 16
    else:
        tr = 8 if ho % 8 == 0 else ho
    n_row_tiles = ho // tr


    cols = jnp.arange(2 * wo + 1)
    rows = jnp.arange(w)[:, None]
    tgt = jnp.where(cols <= wo,
                    jnp.where(cols == 0, 1, 2 * cols - 1),
                    2 * (cols - wo - 1))
    sel = (rows == tgt[None, :]).astype(jnp.float32)

    wt = [[weight[:, :, kh, kw].astype(jnp.bfloat16) for kw in range(3)]
          for kh in range(3)]
    z = jnp.zeros((cout, cin), jnp.bfloat16)
    wall = jnp.stack([
        jnp.concatenate([wt[0][0], wt[0][1]], axis=1),
        jnp.concatenate([wt[1][0], wt[1][1]], axis=1),
        jnp.concatenate([wt[2][0], wt[2][1]], axis=1),
        jnp.concatenate([wt[0][2], wt[1][2]], axis=1),
        jnp.concatenate([wt[2][2], z], axis=1),
    ], axis=0)

    tw = tr * wo
    nr = n_row_tiles
    body = _make_body(cin, cout, ho, wo, tr, nr)
    out = pl.pallas_call(
        body,
        out_shape=jax.ShapeDtypeStruct((n, cout, ho, wo), x.dtype),
        grid=(n, nr + 1),
        in_specs=[
            pl.BlockSpec((1, cin, 2 * tr, w),
                         lambda b, t: (b, 0, jnp.minimum(t, nr - 1), 0)),
            pl.BlockSpec((1, cin, 8, w),
                         lambda b, t: (b, 0, jnp.minimum(
                             (tr // 4) * (jnp.minimum(t, nr - 1) + 1),
                             h // 8 - 1), 0)),
            pl.BlockSpec((w, 2 * wo + 1), lambda b, t: (0, 0)),
            pl.BlockSpec((5, cout, 2 * cin), lambda b, t: (0, 0, 0)),
        ],
        out_specs=pl.BlockSpec((1, cout, ho, wo),
                               lambda b, t: (b, 0, 0, 0)),
        scratch_shapes=[
            pltpu.VMEM((6 * cin, (tr + 1) * wo), jnp.bfloat16),
            pltpu.VMEM((cout, ho * wo), jnp.bfloat16),
            pltpu.VMEM((2, cout, 1), jnp.float32),
            pltpu.VMEM((cin, w), jnp.float32),
        ],
        compiler_params=pltpu.CompilerParams(
            dimension_semantics=("parallel", "arbitrary")),
    )(x, x, sel, wall)
    return out
